# Initial kernel scaffold; baseline (speedup 1.0000x reference)
import jax
import jax.numpy as jnp
from jax.experimental import pallas as pl


def kernel(input_ids, embed_table):
    raise NotImplementedError("write your pallas kernel here")


# speedup vs baseline: 1.4069x; 1.4069x over previous
import functools

import jax
import jax.numpy as jnp
from jax import lax
from jax.experimental import pallas as pl
from jax.experimental.pallas import tpu as pltpu
from jax.experimental.pallas import tpu_sc as plsc

D_MODEL = 2048
NC = 2
NS = 16
NW = NC * NS
CHUNK = 16


def _sc_gather(table, idx3):
    nchunk = idx3.shape[1]
    b_total = NW * nchunk * CHUNK
    mesh = plsc.VectorSubcoreMesh(core_axis_name="c", subcore_axis_name="s")

    @functools.partial(
        pl.kernel,
        out_type=jax.ShapeDtypeStruct((b_total, D_MODEL), jnp.float32),
        mesh=mesh,
        scratch_types=[
            pltpu.VMEM((nchunk, CHUNK), jnp.int32),
            pltpu.VMEM((CHUNK, D_MODEL), jnp.float32),
            pltpu.SemaphoreType.DMA,
        ],
    )
    def k(table_hbm, idx_hbm, out_hbm, idx_v, buf, sem):
        wid = lax.axis_index("s") * NC + lax.axis_index("c")
        base = wid * (nchunk * CHUNK)
        pltpu.sync_copy(idx_hbm.at[wid], idx_v)
        for c in range(nchunk):
            pltpu.async_copy(table_hbm.at[idx_v.at[c]], buf, sem).wait()
            pltpu.sync_copy(buf, out_hbm.at[pl.ds(base + c * CHUNK, CHUNK)])

    return k(table, idx3)


def kernel(input_ids, embed_table):
    ids = input_ids.reshape(NW, -1, CHUNK)
    out = _sc_gather(embed_table, ids)
    return out.reshape(*input_ids.shape, D_MODEL)


# speedup vs baseline: 1.6507x; 1.1733x over previous
import functools

import jax
import jax.numpy as jnp
from jax import lax
from jax.experimental import pallas as pl
from jax.experimental.pallas import tpu as pltpu
from jax.experimental.pallas import tpu_sc as plsc

D_MODEL = 2048
NC = 2
NS = 16
NW = NC * NS
CHUNK = 16
NBUF = 3
LA = NBUF - 1


def _sc_gather(table, idx3):
    nchunk = idx3.shape[1]
    b_total = NW * nchunk * CHUNK
    mesh = plsc.VectorSubcoreMesh(core_axis_name="c", subcore_axis_name="s")

    @functools.partial(
        pl.kernel,
        out_type=jax.ShapeDtypeStruct((b_total, D_MODEL), jnp.float32),
        mesh=mesh,
        scratch_types=[
            pltpu.VMEM((nchunk, CHUNK), jnp.int32),
            pltpu.VMEM((NBUF, CHUNK, D_MODEL), jnp.float32),
            pltpu.SemaphoreType.DMA((NBUF,)),
            pltpu.SemaphoreType.DMA((NBUF,)),
        ],
    )
    def k(table_hbm, idx_hbm, out_hbm, idx_v, bufs, gsem, ssem):
        wid = lax.axis_index("s") * NC + lax.axis_index("c")
        base = wid * (nchunk * CHUNK)
        pltpu.sync_copy(idx_hbm.at[wid], idx_v)
        gath = [None] * nchunk
        outc = [None] * nchunk
        for t in range(nchunk + LA):
            if t < nchunk:
                buf = t % NBUF
                if t >= NBUF:
                    outc[t - NBUF].wait()
                gath[t] = pltpu.async_copy(
                    table_hbm.at[idx_v.at[t]], bufs.at[buf], gsem.at[buf])
            j = t - LA
            if j >= 0:
                gath[j].wait()
                outc[j] = pltpu.async_copy(
                    bufs.at[j % NBUF],
                    out_hbm.at[pl.ds(base + j * CHUNK, CHUNK)],
                    ssem.at[j % NBUF])
        for j in range(nchunk - NBUF, nchunk):
            outc[j].wait()

    return k(table, idx3)


def kernel(input_ids, embed_table):
    ids = input_ids.reshape(NW, -1, CHUNK)
    out = _sc_gather(embed_table, ids)
    return out.reshape(*input_ids.shape, D_MODEL)
